# Initial kernel scaffold; baseline (speedup 1.0000x reference)
import jax
import jax.numpy as jnp
from jax.experimental import pallas as pl


def kernel(query_points, codes, params):
    raise NotImplementedError("write your pallas kernel here")


# speedup vs baseline: 1.9226x; 1.9226x over previous
import jax
import jax.numpy as jnp
from jax.experimental import pallas as pl

G = 512
HID = 128
K = 27
CHUNK = 64
HI = jax.lax.Precision.HIGHEST


def _tables_kernel(codes_ref, w1g_ref, b1_ref, wn1h_ref, bn1_ref, wn2_ref,
                   bn2_ref, t_ref):
    hg = codes_ref[...]
    for l in range(4):
        t_ref[l] = jnp.dot(hg, w1g_ref[l], precision=HI) + b1_ref[l]
        if l < 3:
            npre = jnp.dot(hg, wn1h_ref[l], precision=HI) + bn1_ref[l]
            hd = jax.nn.silu(npre)
            hd = jnp.dot(hd, wn2_ref[l], precision=HI) + bn2_ref[l]
            hg = hg + hd


def _main_kernel(qp_ref, t_ref, w1q_ref, w1d_ref, w2_ref, b2_ref, wc_ref,
                 bc_ref, wn1h_ref, wn1m_ref, bn1_ref, wn2_ref, bn2_ref,
                 out_ref):
    f32 = jnp.float32
    R = CHUNK * K
    qp = qp_ref[...]
    cellf = jnp.floor((qp + 7.0) * 0.5 + 0.5)

    i32 = jnp.int32
    r_f = jax.lax.broadcasted_iota(i32, (R, 1), 0).astype(f32)
    q_of_r = jnp.floor((r_f + 0.5) * (1.0 / 27.0))
    o = r_f - q_of_r * 27.0
    oxp = jnp.floor((o + 0.5) * (1.0 / 9.0))
    o2 = o - oxp * 9.0
    oyp = jnp.floor((o2 + 0.5) * (1.0 / 3.0))
    oz = o2 - oyp * 3.0 - 1.0
    ox = oxp - 1.0
    oy = oyp - 1.0

    qcol = jax.lax.broadcasted_iota(i32, (R, CHUNK), 1).astype(f32)
    bq = (q_of_r == qcol).astype(f32)
    s_q = jax.lax.broadcasted_iota(i32, (CHUNK, R), 0).astype(f32)
    s_r = jax.lax.broadcasted_iota(i32, (CHUNK, R), 1).astype(f32)
    s0 = (jnp.floor((s_r + 0.5) * (1.0 / 27.0)) == s_q).astype(f32)

    pk = jnp.dot(bq, jnp.concatenate([qp, cellf], axis=1), precision=HI)
    qpx, qpy, qpz = pk[:, 0:1], pk[:, 1:2], pk[:, 2:3]
    gx = pk[:, 3:4] + ox
    gy = pk[:, 4:5] + oy
    gz = pk[:, 5:6] + oz
    inb = ((gx >= 0.0) & (gx <= 7.0) & (gy >= 0.0) & (gy <= 7.0)
           & (gz >= 0.0) & (gz <= 7.0))
    gxc = jnp.clip(gx, 0.0, 7.0)
    gyc = jnp.clip(gy, 0.0, 7.0)
    gzc = jnp.clip(gz, 0.0, 7.0)
    nid = gxc * 64.0 + gyc * 8.0 + gzc
    cgx = 2.0 * gxc - 7.0
    cgy = 2.0 * gyc - 7.0
    cgz = 2.0 * gzc - 7.0
    dx0 = qpx - cgx
    dy0 = qpy - cgy
    dz0 = qpz - cgz
    d2 = dx0 * dx0 + dy0 * dy0 + dz0 * dz0
    emask = (inb & (d2 <= 4.0)).astype(f32)

    gcol = jax.lax.broadcasted_iota(i32, (R, G), 1).astype(f32)
    onehot = ((nid == gcol) & inb).astype(f32)
    sel = jnp.concatenate([onehot, bq], axis=1)

    hq = jnp.zeros((CHUNK, HID), f32)
    xq = qp
    for l in range(4):
        xr = jnp.dot(bq, xq, precision=HI)
        relx = cgx - xr[:, 0:1]
        rely = cgy - xr[:, 1:2]
        relz = cgz - xr[:, 2:3]
        dist = jnp.sqrt(relx * relx + rely * rely + relz * relz)
        q1 = jnp.dot(hq, w1q_ref[l], precision=HI)
        table = jnp.concatenate([t_ref[l, 0], q1], axis=0)
        pre = jnp.dot(sel, table, precision=HI) + dist * w1d_ref[l]
        m1 = jax.nn.silu(pre)
        m2 = jax.nn.silu(jnp.dot(m1, w2_ref[l], precision=HI) + b2_ref[l])
        cw = 0.5 * (jnp.cos(dist * (jnp.pi / 2.0)) + 1.0)
        cw = cw * (dist <= 2.0).astype(f32)
        m = m2 * cw
        inv = 1.0 / (dist + 1e-8)
        if l < 3:
            coef = jnp.dot(m, wc_ref[l, :, 0:1], precision=HI) + bc_ref[l, :, 0:1]
            cm = jnp.concatenate(
                [coef * (relx * inv), coef * (rely * inv), coef * (relz * inv)],
                axis=1)
            payload = jnp.concatenate([m * emask, cm * emask, emask], axis=1)
            red = jnp.dot(s0, payload, precision=HI)
            cnt = jnp.maximum(red[:, 131:132], 1.0)
            ma = red[:, 0:128] / cnt
            dxq = red[:, 128:131] / cnt
            xq = xq + dxq
            npre = (jnp.dot(hq, wn1h_ref[l], precision=HI)
                    + jnp.dot(ma, wn1m_ref[l], precision=HI) + bn1_ref[l])
            hd = jax.nn.silu(npre)
            hd = jnp.dot(hd, wn2_ref[l], precision=HI) + bn2_ref[l]
            hq = hq + hd
        else:
            coef = jnp.dot(m, wc_ref[l], precision=HI) + bc_ref[l]
            dirx = relx * inv
            diry = rely * inv
            dirz = relz * inv
            cms = []
            for a in range(5):
                ca = coef[:, a:a + 1]
                cms += [ca * dirx, ca * diry, ca * dirz]
            payload = jnp.concatenate(
                [jnp.concatenate(cms, axis=1) * emask, emask], axis=1)
            red = jnp.dot(s0, payload, precision=HI)
            cnt = jnp.maximum(red[:, 15:16], 1.0)
            dx15 = red[:, 0:15] / cnt
            base = xq - qp
            out_ref[...] = dx15 + jnp.concatenate([base] * 5, axis=1)


def kernel(query_points, codes, params):
    B, N, _ = query_points.shape
    layers = list(params['layers']) + [params['field']]
    f32 = jnp.float32

    w1g = jnp.stack([lp['edge1']['W'][0:128] for lp in layers])
    w1q = jnp.stack([lp['edge1']['W'][128:256] for lp in layers])
    w1d = jnp.stack([lp['edge1']['W'][256:257] for lp in layers])
    b1 = jnp.stack([lp['edge1']['b'][None, :] for lp in layers])
    w2 = jnp.stack([lp['edge2']['W'] for lp in layers])
    b2 = jnp.stack([lp['edge2']['b'][None, :] for lp in layers])
    wc = jnp.stack([
        jnp.pad(lp['coord']['W'], ((0, 0), (0, 5 - lp['coord']['W'].shape[1])))
        for lp in layers])
    bc = jnp.stack([
        jnp.pad(lp['coord']['b'], (0, 5 - lp['coord']['b'].shape[0]))[None, :]
        for lp in layers])
    wn1h = jnp.stack([lp['node1']['W'][0:128] for lp in layers[:3]])
    wn1m = jnp.stack([lp['node1']['W'][128:256] for lp in layers[:3]])
    bn1 = jnp.stack([lp['node1']['b'][None, :] for lp in layers[:3]])
    wn2 = jnp.stack([lp['node2']['W'] for lp in layers[:3]])
    bn2 = jnp.stack([lp['node2']['b'][None, :] for lp in layers[:3]])

    tables = pl.pallas_call(
        _tables_kernel,
        out_shape=jax.ShapeDtypeStruct((4, B * G, HID), f32),
    )(codes.reshape(-1, HID), w1g, b1, wn1h, bn1, wn2, bn2)
    tables = tables.reshape(4, B, G, HID)

    per_b = N // CHUNK
    full = lambda w: pl.BlockSpec(w.shape, lambda i: (0,) * w.ndim)
    out = pl.pallas_call(
        _main_kernel,
        grid=(B * N // CHUNK,),
        in_specs=[
            pl.BlockSpec((CHUNK, 3), lambda i: (i, 0)),
            pl.BlockSpec((4, 1, G, HID), lambda i: (0, i // per_b, 0, 0)),
            full(w1q), full(w1d), full(w2), full(b2), full(wc), full(bc),
            full(wn1h), full(wn1m), full(bn1), full(wn2), full(bn2),
        ],
        out_specs=pl.BlockSpec((CHUNK, 15), lambda i: (i, 0)),
        out_shape=jax.ShapeDtypeStruct((B * N, 15), f32),
    )(query_points.reshape(-1, 3), tables, w1q, w1d, w2, b2, wc, bc,
      wn1h, wn1m, bn1, wn2, bn2)

    return out.reshape(B, N, 5, 3)


# speedup vs baseline: 2.7244x; 1.4170x over previous
import jax
import jax.numpy as jnp
from jax.experimental import pallas as pl
from jax.experimental.pallas import tpu as pltpu

G = 512
HID = 128
K = 27
CHUNK = 128
HI = jax.lax.Precision.HIGHEST
BF = jnp.bfloat16


def _split(x):
    hi = x.astype(BF)
    lo = (x - hi.astype(jnp.float32)).astype(BF)
    return hi, lo


def _tables_kernel(codes_ref, w1g_ref, b1_ref, wn1h_ref, bn1_ref, wn2_ref,
                   bn2_ref, th_ref, tl_ref):
    hg = codes_ref[...]
    for l in range(4):
        t = jnp.dot(hg, w1g_ref[l], precision=HI) + b1_ref[l]
        hi, lo = _split(t)
        th_ref[l] = hi
        tl_ref[l] = lo
        if l < 3:
            npre = jnp.dot(hg, wn1h_ref[l], precision=HI) + bn1_ref[l]
            hd = jax.nn.silu(npre)
            hd = jnp.dot(hd, wn2_ref[l], precision=HI) + bn2_ref[l]
            hg = hg + hd


def _main_kernel(qp_ref, th_ref, tl_ref, w1q_ref, w1d_ref, w2h_ref, w2l_ref,
                 b2_ref, wc_ref, bc_ref, wn1h_ref, wn1m_ref, bn1_ref, wn2_ref,
                 bn2_ref, out_ref):
    f32 = jnp.float32
    i32 = jnp.int32
    R = CHUNK * K
    qp = qp_ref[...]
    cellf = jnp.floor((qp + 7.0) * 0.5 + 0.5)

    r_f = jax.lax.broadcasted_iota(i32, (R, 1), 0).astype(f32)
    q_of_r = jnp.floor((r_f + 0.5) * (1.0 / 27.0))
    o = r_f - q_of_r * 27.0
    oxp = jnp.floor((o + 0.5) * (1.0 / 9.0))
    o2 = o - oxp * 9.0
    oyp = jnp.floor((o2 + 0.5) * (1.0 / 3.0))
    oz = o2 - oyp * 3.0 - 1.0
    ox = oxp - 1.0
    oy = oyp - 1.0

    qcol = jax.lax.broadcasted_iota(i32, (R, CHUNK), 1).astype(f32)
    bq = (q_of_r == qcol).astype(f32)
    bqb = bq.astype(BF)
    s_q = jax.lax.broadcasted_iota(i32, (CHUNK, R), 0).astype(f32)
    s_r = jax.lax.broadcasted_iota(i32, (CHUNK, R), 1).astype(f32)
    s0 = (jnp.floor((s_r + 0.5) * (1.0 / 27.0)) == s_q).astype(BF)

    pk = jnp.dot(bq, jnp.concatenate([qp, cellf], axis=1), precision=HI)
    qpx, qpy, qpz = pk[:, 0:1], pk[:, 1:2], pk[:, 2:3]
    gx = pk[:, 3:4] + ox
    gy = pk[:, 4:5] + oy
    gz = pk[:, 5:6] + oz
    inb = ((gx >= 0.0) & (gx <= 7.0) & (gy >= 0.0) & (gy <= 7.0)
           & (gz >= 0.0) & (gz <= 7.0))
    gxc = jnp.clip(gx, 0.0, 7.0)
    gyc = jnp.clip(gy, 0.0, 7.0)
    gzc = jnp.clip(gz, 0.0, 7.0)
    nid = gxc * 64.0 + gyc * 8.0 + gzc
    cgx = 2.0 * gxc - 7.0
    cgy = 2.0 * gyc - 7.0
    cgz = 2.0 * gzc - 7.0
    dx0 = qpx - cgx
    dy0 = qpy - cgy
    dz0 = qpz - cgz
    d2 = dx0 * dx0 + dy0 * dy0 + dz0 * dz0
    emask = (inb & (d2 <= 4.0)).astype(f32)

    gcol = jax.lax.broadcasted_iota(i32, (R, G), 1).astype(f32)
    onehot = ((nid == gcol) & inb).astype(BF)
    sel = jnp.concatenate([onehot, bqb], axis=1)

    hq = jnp.zeros((CHUNK, HID), f32)
    xq = qp
    for l in range(4):
        xr = jnp.dot(bq, xq, precision=HI)
        relx = cgx - xr[:, 0:1]
        rely = cgy - xr[:, 1:2]
        relz = cgz - xr[:, 2:3]
        dist = jnp.sqrt(relx * relx + rely * rely + relz * relz)
        q1 = jnp.dot(hq, w1q_ref[l], precision=HI)
        q1h, q1l = _split(q1)
        th = jnp.concatenate([th_ref[l, 0], q1h], axis=0)
        tl = jnp.concatenate([tl_ref[l, 0], q1l], axis=0)
        pre = (jnp.dot(sel, th, preferred_element_type=f32)
               + jnp.dot(sel, tl, preferred_element_type=f32)
               + dist * w1d_ref[l])
        m1 = jax.nn.silu(pre)
        m1h, m1l = _split(m1)
        m2p = (jnp.dot(m1h, w2h_ref[l], preferred_element_type=f32)
               + jnp.dot(m1h, w2l_ref[l], preferred_element_type=f32)
               + jnp.dot(m1l, w2h_ref[l], preferred_element_type=f32)
               + b2_ref[l])
        m2 = jax.nn.silu(m2p)
        cw = 0.5 * (jnp.cos(dist * (jnp.pi / 2.0)) + 1.0)
        cw = cw * (dist <= 2.0).astype(f32)
        m = m2 * cw
        inv = 1.0 / (dist + 1e-8)
        if l < 3:
            coef = jnp.dot(m, wc_ref[l, :, 0:1], precision=HI) + bc_ref[l, :, 0:1]
            cm = jnp.concatenate(
                [coef * (relx * inv), coef * (rely * inv), coef * (relz * inv)],
                axis=1)
            payload = jnp.concatenate([m * emask, cm * emask, emask], axis=1)
            ph, plo = _split(payload)
            red = (jnp.dot(s0, ph, preferred_element_type=f32)
                   + jnp.dot(s0, plo, preferred_element_type=f32))
            cnt = jnp.maximum(red[:, 131:132], 1.0)
            ma = red[:, 0:128] / cnt
            dxq = red[:, 128:131] / cnt
            xq = xq + dxq
            npre = (jnp.dot(hq, wn1h_ref[l], precision=HI)
                    + jnp.dot(ma, wn1m_ref[l], precision=HI) + bn1_ref[l])
            hd = jax.nn.silu(npre)
            hd = jnp.dot(hd, wn2_ref[l], precision=HI) + bn2_ref[l]
            hq = hq + hd
        else:
            coef = jnp.dot(m, wc_ref[l], precision=HI) + bc_ref[l]
            dirx = relx * inv
            diry = rely * inv
            dirz = relz * inv
            cms = []
            for a in range(5):
                ca = coef[:, a:a + 1]
                cms += [ca * dirx, ca * diry, ca * dirz]
            payload = jnp.concatenate(
                [jnp.concatenate(cms, axis=1) * emask, emask], axis=1)
            ph, plo = _split(payload)
            red = (jnp.dot(s0, ph, preferred_element_type=f32)
                   + jnp.dot(s0, plo, preferred_element_type=f32))
            cnt = jnp.maximum(red[:, 15:16], 1.0)
            dx15 = red[:, 0:15] / cnt
            base = xq - qp
            out_ref[...] = dx15 + jnp.concatenate([base] * 5, axis=1)


def kernel(query_points, codes, params):
    B, N, _ = query_points.shape
    layers = list(params['layers']) + [params['field']]
    f32 = jnp.float32

    w1g = jnp.stack([lp['edge1']['W'][0:128] for lp in layers])
    w1q = jnp.stack([lp['edge1']['W'][128:256] for lp in layers])
    w1d = jnp.stack([lp['edge1']['W'][256:257] for lp in layers])
    b1 = jnp.stack([lp['edge1']['b'][None, :] for lp in layers])
    w2 = jnp.stack([lp['edge2']['W'] for lp in layers])
    w2h = w2.astype(BF)
    w2l = (w2 - w2h.astype(f32)).astype(BF)
    b2 = jnp.stack([lp['edge2']['b'][None, :] for lp in layers])
    wc = jnp.stack([
        jnp.pad(lp['coord']['W'], ((0, 0), (0, 5 - lp['coord']['W'].shape[1])))
        for lp in layers])
    bc = jnp.stack([
        jnp.pad(lp['coord']['b'], (0, 5 - lp['coord']['b'].shape[0]))[None, :]
        for lp in layers])
    wn1h = jnp.stack([lp['node1']['W'][0:128] for lp in layers[:3]])
    wn1m = jnp.stack([lp['node1']['W'][128:256] for lp in layers[:3]])
    bn1 = jnp.stack([lp['node1']['b'][None, :] for lp in layers[:3]])
    wn2 = jnp.stack([lp['node2']['W'] for lp in layers[:3]])
    bn2 = jnp.stack([lp['node2']['b'][None, :] for lp in layers[:3]])

    th, tl = pl.pallas_call(
        _tables_kernel,
        out_shape=(jax.ShapeDtypeStruct((4, B * G, HID), BF),
                   jax.ShapeDtypeStruct((4, B * G, HID), BF)),
    )(codes.reshape(-1, HID), w1g, b1, wn1h, bn1, wn2, bn2)
    th = th.reshape(4, B, G, HID)
    tl = tl.reshape(4, B, G, HID)

    per_b = N // CHUNK
    full = lambda w: pl.BlockSpec(w.shape, lambda i: (0,) * w.ndim)
    tspec = pl.BlockSpec((4, 1, G, HID), lambda i: (0, i // per_b, 0, 0))
    out = pl.pallas_call(
        _main_kernel,
        grid=(B * N // CHUNK,),
        in_specs=[
            pl.BlockSpec((CHUNK, 3), lambda i: (i, 0)),
            tspec, tspec,
            full(w1q), full(w1d), full(w2h), full(w2l), full(b2), full(wc),
            full(bc), full(wn1h), full(wn1m), full(bn1), full(wn2), full(bn2),
        ],
        out_specs=pl.BlockSpec((CHUNK, 15), lambda i: (i, 0)),
        out_shape=jax.ShapeDtypeStruct((B * N, 15), f32),
        compiler_params=pltpu.CompilerParams(
            dimension_semantics=("parallel",)),
    )(query_points.reshape(-1, 3), th, tl, w1q, w1d, w2h, w2l, b2, wc, bc,
      wn1h, wn1m, bn1, wn2, bn2)

    return out.reshape(B, N, 5, 3)
